# Initial kernel scaffold; baseline (speedup 1.0000x reference)
import jax
import jax.numpy as jnp
from jax.experimental import pallas as pl


def kernel(x_paper, x_author, edge_index_writes, edge_index_cites, W_paper, b_paper, W_author, b_author, W_out, b_out):
    raise NotImplementedError("write your pallas kernel here")


# speedup vs baseline: 7.9149x; 7.9149x over previous
import functools

import jax
import jax.numpy as jnp
from jax import lax
from jax.experimental import pallas as pl
from jax.experimental.pallas import tpu as pltpu
from jax.experimental.pallas import tpu_sc as plsc

N_PAPER = 10000
N_AUTHOR = 10000
D = 256
H = 256
C = 64
E = 160000
ALPHA = 0.01

NC = 2
NS = 16
NW = NC * NS
CHUNK = 128
DUMP = N_PAPER
NROWS_PAD = 10112
ZSTRIPE = NROWS_PAD // NS

_f32 = jnp.float32


_DENSE_BLK = 1000


def _dense_body(x_ref, w_ref, b_ref, wout_ref, o_ref):
    h = jnp.dot(x_ref[...], w_ref[0], preferred_element_type=_f32)
    h = jnp.maximum(h + b_ref[0], 0.0)
    o_ref[...] = jnp.dot(h, wout_ref[...], preferred_element_type=_f32)


def _dense_project(x_stack, w_stack, b_stack, w_out):
    n_total = x_stack.shape[0]
    nblk = n_total // _DENSE_BLK
    per_type = nblk // 2
    return pl.pallas_call(
        _dense_body,
        grid=(nblk,),
        in_specs=[
            pl.BlockSpec((_DENSE_BLK, D), lambda i: (i, 0)),
            pl.BlockSpec((1, D, H), lambda i: (i // per_type, 0, 0)),
            pl.BlockSpec((1, 1, H), lambda i: (i // per_type, 0, 0)),
            pl.BlockSpec((H, C), lambda i: (0, 0)),
        ],
        out_specs=pl.BlockSpec((_DENSE_BLK, C), lambda i: (i, 0)),
        out_shape=jax.ShapeDtypeStruct((n_total, C), _f32),
    )(x_stack, w_stack, b_stack, w_out)


def _segsum_body(nchunks, table, src3, dst3, zeros, out,
                 srcv, dstv, rows, acc):
    c = lax.axis_index("c")
    s = lax.axis_index("s")
    wid = c * NS + s
    pltpu.sync_copy(zeros.at[pl.ds(s * ZSTRIPE, ZSTRIPE)],
                    acc.at[pl.ds(s * ZSTRIPE, ZSTRIPE)])
    pltpu.sync_copy(src3.at[wid], srcv)
    pltpu.sync_copy(dst3.at[wid], dstv)
    plsc.subcore_barrier()

    def body(j, carry):
        pltpu.sync_copy(table.at[srcv.at[j]], rows)
        pltpu.sync_copy(rows, acc.at[dstv.at[j]], add=True)
        return carry

    lax.fori_loop(0, nchunks, body, 0)
    plsc.subcore_barrier()
    pltpu.sync_copy(acc.at[pl.ds(s * ZSTRIPE, ZSTRIPE)],
                    out.at[c, pl.ds(s * ZSTRIPE, ZSTRIPE)])


def _sc_segsum(table, src3, dst3, zeros):
    nchunks = src3.shape[1]
    mesh = plsc.VectorSubcoreMesh(core_axis_name="c", subcore_axis_name="s")
    kern = functools.partial(
        pl.kernel,
        out_type=jax.ShapeDtypeStruct((NC, NROWS_PAD, C), _f32),
        mesh=mesh,
        scratch_types=[
            pltpu.VMEM((nchunks, CHUNK), jnp.int32),
            pltpu.VMEM((nchunks, CHUNK), jnp.int32),
            pltpu.VMEM((CHUNK, C), _f32),
            pltpu.VMEM_SHARED((NROWS_PAD, C), _f32),
        ],
        compiler_params=pltpu.CompilerParams(use_tc_tiling_on_sc=False),
    )(functools.partial(_segsum_body, nchunks))
    return kern(table, src3, dst3, zeros)


_CBLK = 1000


def _add2_body(p_ref, o_ref):
    o_ref[...] = p_ref[0] + p_ref[1]


def _add_partials(parts):
    return pl.pallas_call(
        _add2_body,
        grid=(N_PAPER // _CBLK,),
        in_specs=[pl.BlockSpec((NC, _CBLK, C), lambda i: (0, i, 0))],
        out_specs=pl.BlockSpec((_CBLK, C), lambda i: (i, 0)),
        out_shape=jax.ShapeDtypeStruct((N_PAPER, C), _f32),
    )(parts)


def _final_body(p_ref, mw_ref, q_ref, b_ref, o_ref):
    o_ref[...] = ((ALPHA * ALPHA) * p_ref[...] + (2.0 * ALPHA) * mw_ref[...]
                  + q_ref[0] + q_ref[1] + b_ref[0])


def _final_combine(p, mw, q_parts, b_out):
    return pl.pallas_call(
        _final_body,
        grid=(N_PAPER // _CBLK,),
        in_specs=[
            pl.BlockSpec((_CBLK, C), lambda i: (i, 0)),
            pl.BlockSpec((_CBLK, C), lambda i: (i, 0)),
            pl.BlockSpec((NC, _CBLK, C), lambda i: (0, i, 0)),
            pl.BlockSpec((1, C), lambda i: (0, 0)),
        ],
        out_specs=pl.BlockSpec((_CBLK, C), lambda i: (i, 0)),
        out_shape=jax.ShapeDtypeStruct((N_PAPER, C), _f32),
    )(p, mw, q_parts, b_out.reshape(1, C))


def _pad_edges(src, dst):
    n = src.shape[0]
    per = NW * CHUNK
    n_pad = -(-n // per) * per
    src = jnp.pad(src, (0, n_pad - n))
    dst = jnp.pad(dst, (0, n_pad - n), constant_values=DUMP)
    k = n_pad // per
    return src.reshape(NW, k, CHUNK), dst.reshape(NW, k, CHUNK)


def kernel(x_paper, x_author, edge_index_writes, edge_index_cites,
           W_paper, b_paper, W_author, b_author, W_out, b_out):
    x_stack = jnp.concatenate([x_paper, x_author], axis=0)
    w_stack = jnp.stack([W_paper, W_author], axis=0)
    b_stack = jnp.stack([b_paper, b_author], axis=0).reshape(2, 1, H)
    pa = _dense_project(x_stack, w_stack, b_stack, W_out)
    p = pa[:N_PAPER]

    ws = edge_index_writes[0].astype(jnp.int32) + N_PAPER
    wd = edge_index_writes[1].astype(jnp.int32)
    cs = edge_index_cites[0].astype(jnp.int32)
    cd = edge_index_cites[1].astype(jnp.int32)

    zeros = jnp.zeros((NROWS_PAD, C), _f32)

    src1, dst1 = _pad_edges(jnp.concatenate([ws, cs]), jnp.concatenate([wd, cd]))
    mw_parts = _sc_segsum(pa, src1, dst1, zeros)
    mw = _add_partials(mw_parts)

    src2, dst2 = _pad_edges(cs, cd)
    q_parts = _sc_segsum(mw, src2, dst2, zeros)

    return _final_combine(p, mw, q_parts, b_out)
